# baseline (speedup 1.0000x reference)
import functools

import jax
import jax.numpy as jnp
from jax import lax
from jax.experimental import pallas as pl
from jax.experimental.pallas import tpu as pltpu
from jax.experimental.pallas import tpu_sc as plsc

_N = 10000
_E = 320000
_D = 128
_H = 128
_C = 10
_G = 64
_LAYERS = 3

_NC = 2
_NS = 16
_NW = _NC * _NS
_CHUNK = 64
_BLK = 16
_DEPTH = 4
_NLAUNCH = 2
_NBLK = 10
_EPT = _CHUNK * _BLK * _NBLK
_EPAD = _EPT * _NS * _NLAUNCH
_RPT = 632
_NPAD = _RPT * _NS


def _sc_agg_body(h_hbm, src_hbm, dst_hbm, zero_hbm,
                 out_hbm, sblk, dblk, rows0, rows1, rows2, rows3, agg,
                 semi0, semi1, semr0, semr1, semr2, semr3):
    s = lax.axis_index("s")
    base = s * _RPT

    pltpu.sync_copy(zero_hbm.at[pl.ds(base, _RPT)], agg.at[pl.ds(base, _RPT)])
    plsc.subcore_barrier()

    isems = (semi0, semi1)
    rbufs = (rows0, rows1, rows2, rows3)
    rsems = (semr0, semr1, semr2, semr3)

    def run_pipeline(srcw, dstw, nblk):
        def load_blk(b, buf):
            pltpu.async_copy(srcw.at[b], sblk.at[buf], isems[buf])
            pltpu.async_copy(dstw.at[b], dblk.at[buf], isems[buf])

        def wait_blk(buf):
            pltpu.make_async_copy(srcw.at[0], sblk.at[buf], isems[buf]).wait()
            pltpu.make_async_copy(dstw.at[0], dblk.at[buf], isems[buf]).wait()

        def process_blk(buf):
            sb = sblk.at[buf]
            db = dblk.at[buf]
            for d in range(_DEPTH - 1):
                pltpu.async_copy(h_hbm.at[sb.at[d]], rbufs[d], rsems[d])
            for k in range(_BLK):
                ka = k + _DEPTH - 1
                if ka < _BLK:
                    pltpu.async_copy(h_hbm.at[sb.at[ka]],
                                     rbufs[ka % _DEPTH], rsems[ka % _DEPTH])
                kd = k % _DEPTH
                pltpu.make_async_copy(h_hbm.at[sb.at[k]],
                                      rbufs[kd], rsems[kd]).wait()
                pltpu.sync_copy(rbufs[kd], agg.at[db.at[k]], add=True)

        load_blk(0, 0)

        def outer(b, carry):
            wait_blk(0)
            load_blk(2 * b + 1, 1)
            process_blk(0)
            wait_blk(1)

            @pl.when(2 * b + 2 < nblk)
            def _():
                load_blk(2 * b + 2, 0)

            process_blk(1)
            return carry

        lax.fori_loop(0, nblk // 2, outer, 0)

    run_pipeline(src_hbm.at[s], dst_hbm.at[s], _NBLK)

    plsc.subcore_barrier()
    pltpu.sync_copy(agg.at[pl.ds(base, _RPT)], out_hbm.at[pl.ds(base, _RPT)])


@functools.cache
def _sc_aggregate_fn():
    return pl.kernel(
        _sc_agg_body,
        out_type=jax.ShapeDtypeStruct((_NPAD, _D), jnp.float32),
        mesh=plsc.VectorSubcoreMesh(core_axis_name="c", subcore_axis_name="s",
                                    num_cores=1),
        scratch_types=[
            pltpu.VMEM((2, _BLK, _CHUNK), jnp.int32),
            pltpu.VMEM((2, _BLK, _CHUNK), jnp.int32),
            pltpu.VMEM((_CHUNK, _D), jnp.float32),
            pltpu.VMEM((_CHUNK, _D), jnp.float32),
            pltpu.VMEM((_CHUNK, _D), jnp.float32),
            pltpu.VMEM((_CHUNK, _D), jnp.float32),
            pltpu.VMEM_SHARED((_NPAD, _D), jnp.float32),
            pltpu.SemaphoreType.DMA,
            pltpu.SemaphoreType.DMA,
            pltpu.SemaphoreType.DMA,
            pltpu.SemaphoreType.DMA,
            pltpu.SemaphoreType.DMA,
            pltpu.SemaphoreType.DMA,
        ],
    )


def _masked_mlp(h, p0, p1, w1, b1, w2, b2, gamma, beta):
    a = h + p0 + p1
    t = jnp.maximum(jnp.dot(a, w1, preferred_element_type=jnp.float32) + b1, 0.0)
    y = jnp.maximum(jnp.dot(t, w2, preferred_element_type=jnp.float32) + b2, 0.0)
    rmask = lax.broadcasted_iota(jnp.int32, (_NPAD, 1), 0) < _N
    y = jnp.where(rmask, y, 0.0)
    mean = jnp.sum(y, axis=0, keepdims=True) * (1.0 / _N)
    d = jnp.where(rmask, y - mean, 0.0)
    var = jnp.sum(d * d, axis=0, keepdims=True) * (1.0 / _N)
    hn = (y - mean) * lax.rsqrt(var + 1e-5) * gamma + beta
    return jnp.where(rmask, jnp.maximum(hn, 0.0), 0.0), rmask


def _tc_mlp_body(h_ref, p0_ref, p1_ref, w1_ref, b1_ref, w2_ref, b2_ref,
                 g_ref, bt_ref, o_ref):
    o_ref[...], _ = _masked_mlp(
        h_ref[...], p0_ref[...], p1_ref[...], w1_ref[...], b1_ref[...],
        w2_ref[...], b2_ref[...], g_ref[...], bt_ref[...])


def _tc_final_body(h_ref, p0_ref, p1_ref, w1_ref, b1_ref, w2_ref, b2_ref,
                   g_ref, bt_ref, bcol_ref, wout_ref, bout_ref, o_ref):
    hfin, rmask = _masked_mlp(
        h_ref[...], p0_ref[...], p1_ref[...], w1_ref[...], b1_ref[...],
        w2_ref[...], b2_ref[...], g_ref[...], bt_ref[...])
    lab = lax.broadcasted_iota(jnp.int32, (_NPAD, _G), 1)
    onehot = jnp.where(rmask & (bcol_ref[...] == lab), 1.0, 0.0)
    pooled = lax.dot_general(onehot, hfin, (((0,), (0,)), ((), ())),
                             preferred_element_type=jnp.float32)
    o_ref[...] = (jnp.dot(pooled, wout_ref[...],
                          preferred_element_type=jnp.float32) + bout_ref[...])


_tc_mlp = pl.pallas_call(
    _tc_mlp_body,
    out_shape=jax.ShapeDtypeStruct((_NPAD, _D), jnp.float32),
)

_tc_final = pl.pallas_call(
    _tc_final_body,
    out_shape=jax.ShapeDtypeStruct((_G, _C), jnp.float32),
)


def kernel(x, edge_index, batch, params):
    src = edge_index[0]
    dst = edge_index[1]
    pad_e = _EPAD - _E
    src_f = jnp.concatenate([src, jnp.zeros((pad_e,), jnp.int32)])
    dst_f = jnp.concatenate([dst, jnp.full((pad_e,), _N, jnp.int32)])
    src_p = src_f.reshape(_NLAUNCH, _NS, _NBLK, _BLK, _CHUNK)
    dst_p = dst_f.reshape(_NLAUNCH, _NS, _NBLK, _BLK, _CHUNK)
    zeros_nd = jnp.zeros((_NPAD, _D), jnp.float32)
    bcol = jnp.full((_NPAD, 1), _G, jnp.int32).at[:_N, 0].set(batch)

    h = jnp.zeros((_NPAD, _D), jnp.float32).at[:_N].set(x)
    out = None
    for l in range(_LAYERS):
        p = params["l%d" % l]
        parts = [_sc_aggregate_fn()(h, src_p[i], dst_p[i], zeros_nd)
                 for i in range(_NLAUNCH)]
        args = (h, parts[0], parts[1], p["W1"], p["b1"][None, :], p["W2"],
                p["b2"][None, :], p["gamma"][None, :], p["beta"][None, :])
        if l < _LAYERS - 1:
            h = _tc_mlp(*args)
        else:
            out = _tc_final(*args, bcol, params["Wout"],
                            params["bout"][None, :])
    return out


# speedup vs baseline: 4.2148x; 4.2148x over previous
import functools

import jax
import jax.numpy as jnp
from jax import lax
from jax.experimental import pallas as pl
from jax.experimental.pallas import tpu as pltpu
from jax.experimental.pallas import tpu_sc as plsc

_N = 10000
_E = 320000
_D = 128
_H = 128
_C = 10
_G = 64
_LAYERS = 3

_NC = 2
_NS = 16
_NW = _NC * _NS
_CHUNK = 64
_BLK = 16
_DEPTH = 4
_NBLK0 = 10
_NBLK1 = 10
_EPT0 = _CHUNK * _BLK * _NBLK0
_EPT1 = _CHUNK * _BLK * _NBLK1
_EPAD = (_EPT0 + _EPT1) * _NS
_RPT = 632
_NPAD = _RPT * _NS


def _sc_agg_body(h_hbm, src0_hbm, dst0_hbm, src1_hbm, dst1_hbm, zero_hbm,
                 out_hbm, sblk, dblk, rows0, rows1, rows2, rows3, agg,
                 semi0, semi1, semr0, semr1, semr2, semr3):
    c = lax.axis_index("c")
    s = lax.axis_index("s")
    base = s * _RPT

    pltpu.sync_copy(zero_hbm.at[pl.ds(base, _RPT)], agg.at[pl.ds(base, _RPT)])
    plsc.subcore_barrier()

    isems = (semi0, semi1)
    rbufs = (rows0, rows1, rows2, rows3)
    rsems = (semr0, semr1, semr2, semr3)

    def run_pipeline(srcw, dstw, nblk):
        def load_blk(b, buf):
            pltpu.async_copy(srcw.at[b], sblk.at[buf], isems[buf])
            pltpu.async_copy(dstw.at[b], dblk.at[buf], isems[buf])

        def wait_blk(buf):
            pltpu.make_async_copy(srcw.at[0], sblk.at[buf], isems[buf]).wait()
            pltpu.make_async_copy(dstw.at[0], dblk.at[buf], isems[buf]).wait()

        def process_blk(buf):
            sb = sblk.at[buf]
            db = dblk.at[buf]
            for d in range(_DEPTH - 1):
                pltpu.async_copy(h_hbm.at[sb.at[d]], rbufs[d], rsems[d])
            for k in range(_BLK):
                ka = k + _DEPTH - 1
                if ka < _BLK:
                    pltpu.async_copy(h_hbm.at[sb.at[ka]],
                                     rbufs[ka % _DEPTH], rsems[ka % _DEPTH])
                kd = k % _DEPTH
                pltpu.make_async_copy(h_hbm.at[sb.at[k]],
                                      rbufs[kd], rsems[kd]).wait()
                pltpu.sync_copy(rbufs[kd], agg.at[db.at[k]], add=True)

        load_blk(0, 0)

        def outer(b, carry):
            wait_blk(0)
            load_blk(2 * b + 1, 1)
            process_blk(0)
            wait_blk(1)

            @pl.when(2 * b + 2 < nblk)
            def _():
                load_blk(2 * b + 2, 0)

            process_blk(1)
            return carry

        lax.fori_loop(0, nblk // 2, outer, 0)

    @pl.when(c == 0)
    def _():
        run_pipeline(src0_hbm.at[s], dst0_hbm.at[s], _NBLK0)

    @pl.when(c == 1)
    def _():
        run_pipeline(src1_hbm.at[s], dst1_hbm.at[s], _NBLK1)

    plsc.subcore_barrier()
    pltpu.sync_copy(agg.at[pl.ds(base, _RPT)],
                    out_hbm.at[c].at[pl.ds(base, _RPT)])


@functools.cache
def _sc_aggregate_fn():
    return pl.kernel(
        _sc_agg_body,
        out_type=jax.ShapeDtypeStruct((_NC, _NPAD, _D), jnp.float32),
        mesh=plsc.VectorSubcoreMesh(core_axis_name="c", subcore_axis_name="s"),
        scratch_types=[
            pltpu.VMEM((2, _BLK, _CHUNK), jnp.int32),
            pltpu.VMEM((2, _BLK, _CHUNK), jnp.int32),
            pltpu.VMEM((_CHUNK, _D), jnp.float32),
            pltpu.VMEM((_CHUNK, _D), jnp.float32),
            pltpu.VMEM((_CHUNK, _D), jnp.float32),
            pltpu.VMEM((_CHUNK, _D), jnp.float32),
            pltpu.VMEM_SHARED((_NPAD, _D), jnp.float32),
            pltpu.SemaphoreType.DMA,
            pltpu.SemaphoreType.DMA,
            pltpu.SemaphoreType.DMA,
            pltpu.SemaphoreType.DMA,
            pltpu.SemaphoreType.DMA,
            pltpu.SemaphoreType.DMA,
        ],
    )


def _masked_mlp(h, p0, p1, w1, b1, w2, b2, gamma, beta):
    a = h + p0 + p1
    t = jnp.maximum(jnp.dot(a, w1, preferred_element_type=jnp.float32) + b1, 0.0)
    y = jnp.maximum(jnp.dot(t, w2, preferred_element_type=jnp.float32) + b2, 0.0)
    rmask = lax.broadcasted_iota(jnp.int32, (_NPAD, 1), 0) < _N
    y = jnp.where(rmask, y, 0.0)
    mean = jnp.sum(y, axis=0, keepdims=True) * (1.0 / _N)
    d = jnp.where(rmask, y - mean, 0.0)
    var = jnp.sum(d * d, axis=0, keepdims=True) * (1.0 / _N)
    hn = (y - mean) * lax.rsqrt(var + 1e-5) * gamma + beta
    return jnp.where(rmask, jnp.maximum(hn, 0.0), 0.0), rmask


def _tc_mlp_body(h_ref, p0_ref, p1_ref, w1_ref, b1_ref, w2_ref, b2_ref,
                 g_ref, bt_ref, o_ref):
    o_ref[...], _ = _masked_mlp(
        h_ref[...], p0_ref[...], p1_ref[...], w1_ref[...], b1_ref[...],
        w2_ref[...], b2_ref[...], g_ref[...], bt_ref[...])


def _tc_final_body(h_ref, p0_ref, p1_ref, w1_ref, b1_ref, w2_ref, b2_ref,
                   g_ref, bt_ref, bcol_ref, wout_ref, bout_ref, o_ref):
    hfin, rmask = _masked_mlp(
        h_ref[...], p0_ref[...], p1_ref[...], w1_ref[...], b1_ref[...],
        w2_ref[...], b2_ref[...], g_ref[...], bt_ref[...])
    lab = lax.broadcasted_iota(jnp.int32, (_NPAD, _G), 1)
    onehot = jnp.where(rmask & (bcol_ref[...] == lab), 1.0, 0.0)
    pooled = lax.dot_general(onehot, hfin, (((0,), (0,)), ((), ())),
                             preferred_element_type=jnp.float32)
    o_ref[...] = (jnp.dot(pooled, wout_ref[...],
                          preferred_element_type=jnp.float32) + bout_ref[...])


_tc_mlp = pl.pallas_call(
    _tc_mlp_body,
    out_shape=jax.ShapeDtypeStruct((_NPAD, _D), jnp.float32),
)

_tc_final = pl.pallas_call(
    _tc_final_body,
    out_shape=jax.ShapeDtypeStruct((_G, _C), jnp.float32),
)


def kernel(x, edge_index, batch, params):
    src = edge_index[0]
    dst = edge_index[1]
    pad_e = _EPAD - _E
    pad_src = (jnp.arange(pad_e, dtype=jnp.int32) * 13) % _N
    pad_dst = _N + (jnp.arange(pad_e, dtype=jnp.int32) % (_NPAD - _N))
    src_f = jnp.concatenate([src, pad_src])
    dst_f = jnp.concatenate([dst, pad_dst])
    n0 = _NS * _EPT0
    src0 = src_f[:n0].reshape(_NS, _NBLK0, _BLK, _CHUNK)
    dst0 = dst_f[:n0].reshape(_NS, _NBLK0, _BLK, _CHUNK)
    src1 = src_f[n0:].reshape(_NS, _NBLK1, _BLK, _CHUNK)
    dst1 = dst_f[n0:].reshape(_NS, _NBLK1, _BLK, _CHUNK)
    zeros_nd = jnp.zeros((_NPAD, _D), jnp.float32)
    bcol = jnp.full((_NPAD, 1), _G, jnp.int32).at[:_N, 0].set(batch)

    h = jnp.zeros((_NPAD, _D), jnp.float32).at[:_N].set(x)
    out = None
    for l in range(_LAYERS):
        p = params["l%d" % l]
        parts = _sc_aggregate_fn()(h, src0, dst0, src1, dst1, zeros_nd)
        args = (h, parts[0], parts[1], p["W1"], p["b1"][None, :], p["W2"],
                p["b2"][None, :], p["gamma"][None, :], p["beta"][None, :])
        if l < _LAYERS - 1:
            h = _tc_mlp(*args)
        else:
            out = _tc_final(*args, bcol, params["Wout"],
                            params["bout"][None, :])
    return out


# speedup vs baseline: 4.5994x; 1.0912x over previous
import functools

import jax
import jax.numpy as jnp
from jax import lax
from jax.experimental import pallas as pl
from jax.experimental.pallas import tpu as pltpu
from jax.experimental.pallas import tpu_sc as plsc

_N = 10000
_E = 320000
_D = 128
_H = 128
_C = 10
_G = 64
_LAYERS = 3

_NC = 2
_NS = 16
_NW = _NC * _NS
_CHUNK = 64
_BLK = 16
_DEPTH = 4
_NBLK0 = 10
_NBLK1 = 10
_EPT0 = _CHUNK * _BLK * _NBLK0
_EPT1 = _CHUNK * _BLK * _NBLK1
_EPAD = (_EPT0 + _EPT1) * _NS
_RPT = 632
_NPAD = _RPT * _NS


def _sc_agg_body(h_hbm, src0_hbm, dst0_hbm, src1_hbm, dst1_hbm,
                 out_hbm, sblk, dblk, rows0, rows1, rows2, rows3, agg,
                 semi0, semi1, semr0, semr1, semr2, semr3):
    c = lax.axis_index("c")
    s = lax.axis_index("s")
    base = s * _RPT

    def zrow(i, carry):
        for j in range(_D // 16):
            rows0[i, pl.ds(j * 16, 16)] = jnp.zeros((16,), jnp.float32)
        return carry

    lax.fori_loop(0, _CHUNK, zrow, 0)
    for t in range(_RPT // _CHUNK):
        pltpu.sync_copy(rows0, agg.at[pl.ds(base + t * _CHUNK, _CHUNK)])
    rem = _RPT % _CHUNK
    if rem:
        pltpu.sync_copy(rows0.at[pl.ds(0, rem)],
                        agg.at[pl.ds(base + _RPT - rem, rem)])
    plsc.subcore_barrier()

    isems = (semi0, semi1)
    rbufs = (rows0, rows1, rows2, rows3)
    rsems = (semr0, semr1, semr2, semr3)

    def run_pipeline(srcw, dstw, nblk):
        def load_blk(b, buf):
            pltpu.async_copy(srcw.at[b], sblk.at[buf], isems[buf])
            pltpu.async_copy(dstw.at[b], dblk.at[buf], isems[buf])

        def wait_blk(buf):
            pltpu.make_async_copy(srcw.at[0], sblk.at[buf], isems[buf]).wait()
            pltpu.make_async_copy(dstw.at[0], dblk.at[buf], isems[buf]).wait()

        def process_blk(buf):
            sb = sblk.at[buf]
            db = dblk.at[buf]
            for d in range(_DEPTH - 1):
                pltpu.async_copy(h_hbm.at[sb.at[d]], rbufs[d], rsems[d])
            for k in range(_BLK):
                ka = k + _DEPTH - 1
                if ka < _BLK:
                    pltpu.async_copy(h_hbm.at[sb.at[ka]],
                                     rbufs[ka % _DEPTH], rsems[ka % _DEPTH])
                kd = k % _DEPTH
                pltpu.make_async_copy(h_hbm.at[sb.at[k]],
                                      rbufs[kd], rsems[kd]).wait()
                pltpu.sync_copy(rbufs[kd], agg.at[db.at[k]], add=True)

        load_blk(0, 0)

        def outer(b, carry):
            wait_blk(0)
            load_blk(2 * b + 1, 1)
            process_blk(0)
            wait_blk(1)

            @pl.when(2 * b + 2 < nblk)
            def _():
                load_blk(2 * b + 2, 0)

            process_blk(1)
            return carry

        lax.fori_loop(0, nblk // 2, outer, 0)

    @pl.when(c == 0)
    def _():
        run_pipeline(src0_hbm.at[s], dst0_hbm.at[s], _NBLK0)

    @pl.when(c == 1)
    def _():
        run_pipeline(src1_hbm.at[s], dst1_hbm.at[s], _NBLK1)

    plsc.subcore_barrier()
    pltpu.sync_copy(agg.at[pl.ds(base, _RPT)],
                    out_hbm.at[c].at[pl.ds(base, _RPT)])


@functools.cache
def _sc_aggregate_fn():
    return pl.kernel(
        _sc_agg_body,
        out_type=jax.ShapeDtypeStruct((_NC, _NPAD, _D), jnp.float32),
        mesh=plsc.VectorSubcoreMesh(core_axis_name="c", subcore_axis_name="s"),
        scratch_types=[
            pltpu.VMEM((2, _BLK, _CHUNK), jnp.int32),
            pltpu.VMEM((2, _BLK, _CHUNK), jnp.int32),
            pltpu.VMEM((_CHUNK, _D), jnp.float32),
            pltpu.VMEM((_CHUNK, _D), jnp.float32),
            pltpu.VMEM((_CHUNK, _D), jnp.float32),
            pltpu.VMEM((_CHUNK, _D), jnp.float32),
            pltpu.VMEM_SHARED((_NPAD, _D), jnp.float32),
            pltpu.SemaphoreType.DMA,
            pltpu.SemaphoreType.DMA,
            pltpu.SemaphoreType.DMA,
            pltpu.SemaphoreType.DMA,
            pltpu.SemaphoreType.DMA,
            pltpu.SemaphoreType.DMA,
        ],
    )


def _mlp(h, p_ref, w1, b1, w2, b2, gamma, beta):
    a = h + p_ref[0, :_N, :] + p_ref[1, :_N, :]
    t = jnp.maximum(jnp.dot(a, w1, preferred_element_type=jnp.float32) + b1, 0.0)
    y = jnp.maximum(jnp.dot(t, w2, preferred_element_type=jnp.float32) + b2, 0.0)
    mean = jnp.sum(y, axis=0, keepdims=True) * (1.0 / _N)
    d = y - mean
    var = jnp.sum(d * d, axis=0, keepdims=True) * (1.0 / _N)
    hn = d * lax.rsqrt(var + 1e-5) * gamma + beta
    return jnp.maximum(hn, 0.0)


def _tc_mlp_body(h_ref, p_ref, w1_ref, b1_ref, w2_ref, b2_ref,
                 g_ref, bt_ref, o_ref):
    o_ref[...] = _mlp(
        h_ref[...], p_ref, w1_ref[...], b1_ref[...],
        w2_ref[...], b2_ref[...], g_ref[...], bt_ref[...])


def _tc_final_body(h_ref, p_ref, w1_ref, b1_ref, w2_ref, b2_ref,
                   g_ref, bt_ref, bcol_ref, wout_ref, bout_ref, o_ref):
    hfin = _mlp(
        h_ref[...], p_ref, w1_ref[...], b1_ref[...],
        w2_ref[...], b2_ref[...], g_ref[...], bt_ref[...])
    lab = lax.broadcasted_iota(jnp.int32, (_N, _G), 1)
    onehot = jnp.where(bcol_ref[...] == lab, 1.0, 0.0)
    pooled = lax.dot_general(onehot, hfin, (((0,), (0,)), ((), ())),
                             preferred_element_type=jnp.float32)
    o_ref[...] = (jnp.dot(pooled, wout_ref[...],
                          preferred_element_type=jnp.float32) + bout_ref[...])


_tc_mlp = pl.pallas_call(
    _tc_mlp_body,
    out_shape=jax.ShapeDtypeStruct((_N, _D), jnp.float32),
)

_tc_final = pl.pallas_call(
    _tc_final_body,
    out_shape=jax.ShapeDtypeStruct((_G, _C), jnp.float32),
)


def kernel(x, edge_index, batch, params):
    src = edge_index[0]
    dst = edge_index[1]
    pad_e = _EPAD - _E
    pad_src = (jnp.arange(pad_e, dtype=jnp.int32) * 13) % _N
    pad_dst = _N + (jnp.arange(pad_e, dtype=jnp.int32) % (_NPAD - _N))
    src_f = jnp.concatenate([src, pad_src])
    dst_f = jnp.concatenate([dst, pad_dst])
    del pad_e
    n0 = _NS * _EPT0
    src0 = src_f[:n0].reshape(_NS, _NBLK0, _BLK, _CHUNK)
    dst0 = dst_f[:n0].reshape(_NS, _NBLK0, _BLK, _CHUNK)
    src1 = src_f[n0:].reshape(_NS, _NBLK1, _BLK, _CHUNK)
    dst1 = dst_f[n0:].reshape(_NS, _NBLK1, _BLK, _CHUNK)
    bcol = batch[:, None]

    h = x
    out = None
    for l in range(_LAYERS):
        p = params["l%d" % l]
        parts = _sc_aggregate_fn()(h, src0, dst0, src1, dst1)
        args = (h, parts, p["W1"], p["b1"][None, :], p["W2"],
                p["b2"][None, :], p["gamma"][None, :], p["beta"][None, :])
        if l < _LAYERS - 1:
            h = _tc_mlp(*args)
        else:
            out = _tc_final(*args, bcol, params["Wout"],
                            params["bout"][None, :])
    return out
